# baseline (speedup 1.0000x reference)
import functools

import jax
import jax.numpy as jnp
from jax import lax
from jax.experimental import pallas as pl
from jax.experimental.pallas import tpu as pltpu
from jax.experimental.pallas import tpu_sc as plsc

_BT = 1024


def _table_kernel(emb_ref, w_ref, bpad_ref, a_ref, carry_ref):
    i = pl.program_id(0)

    @pl.when(i == 0)
    def _():
        carry_ref[...] = jnp.zeros_like(carry_ref)

    h = emb_ref.shape[1]
    emb = emb_ref[...]
    nd = (((1,), (1,)), ((), ()))
    p1 = lax.dot_general(emb, w_ref[:, 0:h], nd,
                         preferred_element_type=jnp.float32)
    p2 = lax.dot_general(emb, w_ref[:, h : 2 * h], nd,
                         preferred_element_type=jnp.float32)
    p3 = lax.dot_general(emb, w_ref[:, 2 * h : 3 * h], nd,
                         preferred_element_type=jnp.float32)
    nl = p1.shape[1]
    bt = p1.shape[0]
    sb = 128
    r = lax.broadcasted_iota(jnp.int32, (sb, sb), 0)
    c = lax.broadcasted_iota(jnp.int32, (sb, sb), 1)
    tri = (r >= c).astype(jnp.float32)
    a_ref[:, 0:nl] = p1 + bpad_ref[0:1, 0:nl]
    a_ref[:, 64 : 64 + nl] = p2
    off = carry_ref[0:1, 0:nl]
    for q in range(bt // sb):
        p3q = p3[q * sb : (q + 1) * sb, :]
        cq = jnp.dot(tri, p3q, preferred_element_type=jnp.float32) + off
        a_ref[q * sb : (q + 1) * sb, 16 : 16 + nl] = p3q - cq
        a_ref[q * sb : (q + 1) * sb, 80 : 80 + nl] = cq
        off = cq[sb - 1 :, :]
    carry_ref[0:1, 0:nl] = off


def _build_table(token_emb, W, bpad):
    t, h = token_emb.shape
    nl = W.shape[0]
    grid = t // _BT
    return pl.pallas_call(
        _table_kernel,
        grid=(grid,),
        in_specs=[
            pl.BlockSpec((_BT, h), lambda i: (i, 0)),
            pl.BlockSpec((nl, 3 * h), lambda i: (0, 0)),
            pl.BlockSpec((8, 128), lambda i: (0, 0)),
        ],
        out_specs=pl.BlockSpec((_BT, 128), lambda i: (i, 0)),
        out_shape=jax.ShapeDtypeStruct((t, 128), jnp.float32),
        scratch_shapes=[pltpu.VMEM((8, 128), jnp.float32)],
        compiler_params=pltpu.CompilerParams(
            dimension_semantics=("arbitrary",)
        ),
    )(token_emb, W, bpad)


_SB = 128


def _make_sc_combine(n, num_logits):
    info = plsc.get_sparse_core_info()
    nc, ns = info.num_cores, info.num_subcores
    nw = nc * ns
    per_w = n // nw
    k_steps = per_w // _SB
    mesh = plsc.VectorSubcoreMesh(core_axis_name="c", subcore_axis_name="s")

    @functools.partial(
        pl.kernel,
        mesh=mesh,
        out_type=jax.ShapeDtypeStruct((n, 128), jnp.float32),
        scratch_types=[
            [pltpu.VMEM((_SB,), jnp.int32)] * 2,
            [pltpu.VMEM((_SB,), jnp.int32)] * 2,
            [pltpu.VMEM((_SB,), jnp.float32)] * 2,
            [pltpu.VMEM((_SB, 128), jnp.float32)] * 2,
            [pltpu.VMEM((_SB, 128), jnp.float32)] * 2,
            [pltpu.VMEM((_SB, 128), jnp.float32)] * 2,
            [pltpu.SemaphoreType.DMA] * 2,
            [pltpu.SemaphoreType.DMA] * 2,
        ],
        compiler_params=pltpu.CompilerParams(use_tc_tiling_on_sc=False),
    )
    def sc_combine(ta, spans_t, out_hbm, sv, em1, invr, ar, br,
                   outv, sem_a, sem_b):
        wid = lax.axis_index("s") * nc + lax.axis_index("c")

        def stage(k, p):
            base = wid * per_w + k * _SB
            pltpu.sync_copy(spans_t.at[0, pl.ds(base, _SB)], sv[p])
            pltpu.sync_copy(spans_t.at[1, pl.ds(base, _SB)], em1[p])
            for g in range(_SB // 16):
                svv = sv[p][pl.ds(g * 16, 16)]
                evv = em1[p][pl.ds(g * 16, 16)]
                em1[p][pl.ds(g * 16, 16)] = evv - 1
                invr[p][pl.ds(g * 16, 16)] = 1.0 / (evv - svv).astype(
                    jnp.float32
                )
            pltpu.async_copy(ta.at[sv[p]], ar[p], sem_a[p])
            pltpu.async_copy(ta.at[em1[p]], br[p], sem_b[p])

        def finish(k, p):
            base = wid * per_w + k * _SB
            pltpu.make_async_copy(ta.at[sv[p]], ar[p], sem_a[p]).wait()
            pltpu.make_async_copy(ta.at[em1[p]], br[p], sem_b[p]).wait()
            for g in range(_SB // 16):
                invv = invr[p][pl.ds(g * 16, 16)]
                for u in range(16):
                    i = g * 16 + u
                    alo = ar[p][i, pl.ds(0, 16)]
                    ahi = ar[p][i, pl.ds(16, 16)]
                    blo = br[p][i, pl.ds(64, 16)]
                    bhi = br[p][i, pl.ds(80, 16)]
                    outv[p][i, pl.ds(0, 16)] = (
                        alo + blo + (bhi + ahi) * invv[u]
                    )
            pltpu.sync_copy(outv[p], out_hbm.at[pl.ds(base, _SB)])

        stage(0, 0)

        def body(g, carry):
            k0 = 2 * g
            stage(k0 + 1, 1)
            finish(k0, 0)

            @pl.when(g < k_steps // 2 - 1)
            def _():
                stage(k0 + 2, 0)

            finish(k0 + 1, 1)
            return carry

        lax.fori_loop(0, k_steps // 2, body, 0)

    return sc_combine


def kernel(token_emb, spans, W, b):
    t, h = token_emb.shape
    n = spans.shape[0]
    num_logits = W.shape[0]

    bpad = jnp.zeros((8, 128), jnp.float32).at[0, 0:num_logits].set(b)

    tab = _build_table(token_emb, W.astype(jnp.float32), bpad)

    spans_t = spans.astype(jnp.int32).T

    out = _make_sc_combine(n, num_logits)(tab, spans_t)
    return out[:, 0:num_logits]


# speedup vs baseline: 1.0595x; 1.0595x over previous
import functools

import jax
import jax.numpy as jnp
from jax import lax
from jax.experimental import pallas as pl
from jax.experimental.pallas import tpu as pltpu
from jax.experimental.pallas import tpu_sc as plsc

_BT = 2048


def _table_kernel(emb_ref, w_ref, bpad_ref, a_ref, carry_ref):
    i = pl.program_id(0)

    @pl.when(i == 0)
    def _():
        carry_ref[...] = jnp.zeros_like(carry_ref)

    h = emb_ref.shape[1]
    emb = emb_ref[...]
    nd = (((1,), (1,)), ((), ()))
    p1 = lax.dot_general(emb, w_ref[:, 0:h], nd,
                         preferred_element_type=jnp.float32)
    p2 = lax.dot_general(emb, w_ref[:, h : 2 * h], nd,
                         preferred_element_type=jnp.float32)
    p3 = lax.dot_general(emb, w_ref[:, 2 * h : 3 * h], nd,
                         preferred_element_type=jnp.float32)
    nl = p1.shape[1]
    bt = p1.shape[0]
    sb = 128
    r = lax.broadcasted_iota(jnp.int32, (sb, sb), 0)
    c = lax.broadcasted_iota(jnp.int32, (sb, sb), 1)
    tri = (r >= c).astype(jnp.float32)
    a_ref[:, 0:nl] = p1 + bpad_ref[0:1, 0:nl]
    a_ref[:, 64 : 64 + nl] = p2
    off = carry_ref[0:1, 0:nl]
    for q in range(bt // sb):
        p3q = p3[q * sb : (q + 1) * sb, :]
        cq = jnp.dot(tri, p3q, preferred_element_type=jnp.float32) + off
        a_ref[q * sb : (q + 1) * sb, 16 : 16 + nl] = p3q - cq
        a_ref[q * sb : (q + 1) * sb, 80 : 80 + nl] = cq
        off = cq[sb - 1 :, :]
    carry_ref[0:1, 0:nl] = off


def _build_table(token_emb, W, bpad):
    t, h = token_emb.shape
    nl = W.shape[0]
    grid = t // _BT
    return pl.pallas_call(
        _table_kernel,
        grid=(grid,),
        in_specs=[
            pl.BlockSpec((_BT, h), lambda i: (i, 0)),
            pl.BlockSpec((nl, 3 * h), lambda i: (0, 0)),
            pl.BlockSpec((8, 128), lambda i: (0, 0)),
        ],
        out_specs=pl.BlockSpec((_BT, 128), lambda i: (i, 0)),
        out_shape=jax.ShapeDtypeStruct((t, 128), jnp.float32),
        scratch_shapes=[pltpu.VMEM((8, 128), jnp.float32)],
        compiler_params=pltpu.CompilerParams(
            dimension_semantics=("arbitrary",)
        ),
    )(token_emb, W, bpad)


_SB = 128


def _make_sc_combine(n, num_logits):
    info = plsc.get_sparse_core_info()
    nc, ns = info.num_cores, info.num_subcores
    nw = nc * ns
    per_w = n // nw
    k_steps = per_w // _SB
    mesh = plsc.VectorSubcoreMesh(core_axis_name="c", subcore_axis_name="s")

    @functools.partial(
        pl.kernel,
        mesh=mesh,
        out_type=jax.ShapeDtypeStruct((n, 128), jnp.float32),
        scratch_types=[
            [pltpu.VMEM((_SB,), jnp.int32)] * 2,
            [pltpu.VMEM((_SB,), jnp.int32)] * 2,
            [pltpu.VMEM((_SB,), jnp.float32)] * 2,
            [pltpu.VMEM((_SB, 128), jnp.float32)] * 2,
            [pltpu.VMEM((_SB, 128), jnp.float32)] * 2,
            [pltpu.VMEM((_SB, 128), jnp.float32)] * 2,
            [pltpu.SemaphoreType.DMA] * 2,
            [pltpu.SemaphoreType.DMA] * 2,
        ],
        compiler_params=pltpu.CompilerParams(use_tc_tiling_on_sc=False),
    )
    def sc_combine(ta, spans_t, out_hbm, sv, em1, invr, ar, br,
                   outv, sem_a, sem_b):
        wid = lax.axis_index("s") * nc + lax.axis_index("c")

        def stage(k, p):
            base = wid * per_w + k * _SB
            pltpu.sync_copy(spans_t.at[0, pl.ds(base, _SB)], sv[p])
            pltpu.sync_copy(spans_t.at[1, pl.ds(base, _SB)], em1[p])
            for g in range(_SB // 16):
                svv = sv[p][pl.ds(g * 16, 16)]
                evv = em1[p][pl.ds(g * 16, 16)]
                em1[p][pl.ds(g * 16, 16)] = evv - 1
                invr[p][pl.ds(g * 16, 16)] = 1.0 / (evv - svv).astype(
                    jnp.float32
                )
            pltpu.async_copy(ta.at[sv[p]], ar[p], sem_a[p])
            pltpu.async_copy(ta.at[em1[p]], br[p], sem_b[p])

        def finish(k, p):
            base = wid * per_w + k * _SB
            pltpu.make_async_copy(ta.at[sv[p]], ar[p], sem_a[p]).wait()
            pltpu.make_async_copy(ta.at[em1[p]], br[p], sem_b[p]).wait()
            for g in range(_SB // 16):
                invv = invr[p][pl.ds(g * 16, 16)]
                for u in range(16):
                    i = g * 16 + u
                    alo = ar[p][i, pl.ds(0, 16)]
                    ahi = ar[p][i, pl.ds(16, 16)]
                    blo = br[p][i, pl.ds(64, 16)]
                    bhi = br[p][i, pl.ds(80, 16)]
                    outv[p][i, pl.ds(0, 16)] = (
                        alo + blo + (bhi + ahi) * invv[u]
                    )
            pltpu.sync_copy(outv[p], out_hbm.at[pl.ds(base, _SB)])

        stage(0, 0)

        def body(g, carry):
            k0 = 2 * g
            stage(k0 + 1, 1)
            finish(k0, 0)

            @pl.when(g < k_steps // 2 - 1)
            def _():
                stage(k0 + 2, 0)

            finish(k0 + 1, 1)
            return carry

        lax.fori_loop(0, k_steps // 2, body, 0)

    return sc_combine


def kernel(token_emb, spans, W, b):
    t, h = token_emb.shape
    n = spans.shape[0]
    num_logits = W.shape[0]

    bpad = jnp.zeros((8, 128), jnp.float32).at[0, 0:num_logits].set(b)

    tab = _build_table(token_emb, W.astype(jnp.float32), bpad)

    spans_t = spans.astype(jnp.int32).T

    out = _make_sc_combine(n, num_logits)(tab, spans_t)
    return out[:, 0:num_logits]


# speedup vs baseline: 1.1401x; 1.0761x over previous
import functools

import jax
import jax.numpy as jnp
from jax import lax
from jax.experimental import pallas as pl
from jax.experimental.pallas import tpu as pltpu
from jax.experimental.pallas import tpu_sc as plsc

_BT = 2048


def _table_kernel(emb_ref, w_ref, bpad_ref, a_ref, carry_ref):
    i = pl.program_id(0)

    @pl.when(i == 0)
    def _():
        carry_ref[...] = jnp.zeros_like(carry_ref)

    h = emb_ref.shape[1]
    emb = emb_ref[...]
    nd = (((1,), (1,)), ((), ()))
    p1 = lax.dot_general(emb, w_ref[:, 0:h], nd,
                         preferred_element_type=jnp.float32)
    p2 = lax.dot_general(emb, w_ref[:, h : 2 * h], nd,
                         preferred_element_type=jnp.float32)
    p3 = lax.dot_general(emb, w_ref[:, 2 * h : 3 * h], nd,
                         preferred_element_type=jnp.float32)
    nl = p1.shape[1]
    bt = p1.shape[0]
    sb = 128
    r = lax.broadcasted_iota(jnp.int32, (sb, sb), 0)
    c = lax.broadcasted_iota(jnp.int32, (sb, sb), 1)
    tri = (r >= c).astype(jnp.float32)
    a_ref[:, 0:nl] = p1 + bpad_ref[0:1, 0:nl]
    a_ref[:, 64 : 64 + nl] = p2
    off = carry_ref[0:1, 0:nl]
    for q in range(bt // sb):
        p3q = p3[q * sb : (q + 1) * sb, :]
        cq = jnp.dot(tri, p3q, preferred_element_type=jnp.float32) + off
        a_ref[q * sb : (q + 1) * sb, 16 : 16 + nl] = p3q - cq
        a_ref[q * sb : (q + 1) * sb, 80 : 80 + nl] = cq
        off = cq[sb - 1 :, :]
    carry_ref[0:1, 0:nl] = off


def _build_table(token_emb, W, bpad):
    t, h = token_emb.shape
    nl = W.shape[0]
    grid = t // _BT
    return pl.pallas_call(
        _table_kernel,
        grid=(grid,),
        in_specs=[
            pl.BlockSpec((_BT, h), lambda i: (i, 0)),
            pl.BlockSpec((nl, 3 * h), lambda i: (0, 0)),
            pl.BlockSpec((8, 128), lambda i: (0, 0)),
        ],
        out_specs=pl.BlockSpec((_BT, 128), lambda i: (i, 0)),
        out_shape=jax.ShapeDtypeStruct((t, 128), jnp.float32),
        scratch_shapes=[pltpu.VMEM((8, 128), jnp.float32)],
        compiler_params=pltpu.CompilerParams(
            dimension_semantics=("arbitrary",)
        ),
    )(token_emb, W, bpad)


_SB = 128


def _make_sc_combine(n, num_logits):
    info = plsc.get_sparse_core_info()
    nc, ns = info.num_cores, info.num_subcores
    nw = nc * ns
    per_w = n // nw
    k_steps = per_w // _SB
    mesh = plsc.VectorSubcoreMesh(core_axis_name="c", subcore_axis_name="s")

    @functools.partial(
        pl.kernel,
        mesh=mesh,
        out_type=jax.ShapeDtypeStruct((n, 128), jnp.float32),
        scratch_types=[
            pltpu.VMEM((per_w,), jnp.int32),
            pltpu.VMEM((per_w,), jnp.int32),
            pltpu.VMEM((per_w,), jnp.float32),
            [pltpu.VMEM((_SB, 128), jnp.float32)] * 2,
            [pltpu.VMEM((_SB, 128), jnp.float32)] * 2,
            [pltpu.VMEM((_SB, 128), jnp.float32)] * 2,
            [pltpu.SemaphoreType.DMA] * 2,
            [pltpu.SemaphoreType.DMA] * 2,
        ],
        compiler_params=pltpu.CompilerParams(use_tc_tiling_on_sc=False),
    )
    def sc_combine(ta, spans_t, out_hbm, sv, em1, invr, ar, br,
                   outv, sem_a, sem_b):
        wid = lax.axis_index("s") * nc + lax.axis_index("c")

        pltpu.sync_copy(spans_t.at[0, pl.ds(wid * per_w, per_w)], sv)
        pltpu.sync_copy(spans_t.at[1, pl.ds(wid * per_w, per_w)], em1)
        for g in range(per_w // 16):
            svv = sv[pl.ds(g * 16, 16)]
            evv = em1[pl.ds(g * 16, 16)]
            em1[pl.ds(g * 16, 16)] = evv - 1
            invr[pl.ds(g * 16, 16)] = 1.0 / (evv - svv).astype(jnp.float32)

        def stage(k, p):
            pltpu.async_copy(
                ta.at[sv.at[pl.ds(k * _SB, _SB)]], ar[p], sem_a[p]
            )
            pltpu.async_copy(
                ta.at[em1.at[pl.ds(k * _SB, _SB)]], br[p], sem_b[p]
            )

        def finish(k, p):
            base = wid * per_w + k * _SB
            pltpu.make_async_copy(
                ta.at[sv.at[pl.ds(k * _SB, _SB)]], ar[p], sem_a[p]
            ).wait()
            pltpu.make_async_copy(
                ta.at[em1.at[pl.ds(k * _SB, _SB)]], br[p], sem_b[p]
            ).wait()
            for g in range(_SB // 16):
                invv = invr[pl.ds(k * _SB + g * 16, 16)]
                for u in range(16):
                    i = g * 16 + u
                    alo = ar[p][i, pl.ds(0, 16)]
                    ahi = ar[p][i, pl.ds(16, 16)]
                    blo = br[p][i, pl.ds(64, 16)]
                    bhi = br[p][i, pl.ds(80, 16)]
                    outv[p][i, pl.ds(0, 16)] = (
                        alo + blo + (bhi + ahi) * invv[u]
                    )
            pltpu.sync_copy(outv[p], out_hbm.at[pl.ds(base, _SB)])

        stage(0, 0)

        def body(g, carry):
            k0 = 2 * g
            stage(k0 + 1, 1)
            finish(k0, 0)

            @pl.when(g < k_steps // 2 - 1)
            def _():
                stage(k0 + 2, 0)

            finish(k0 + 1, 1)
            return carry

        lax.fori_loop(0, k_steps // 2, body, 0)

    return sc_combine


def kernel(token_emb, spans, W, b):
    t, h = token_emb.shape
    n = spans.shape[0]
    num_logits = W.shape[0]

    bpad = jnp.zeros((8, 128), jnp.float32).at[0, 0:num_logits].set(b)

    tab = _build_table(token_emb, W.astype(jnp.float32), bpad)

    spans_t = spans.astype(jnp.int32).T

    out = _make_sc_combine(n, num_logits)(tab, spans_t)
    return out[:, 0:num_logits]


# speedup vs baseline: 1.1451x; 1.0044x over previous
import functools

import jax
import jax.numpy as jnp
from jax import lax
from jax.experimental import pallas as pl
from jax.experimental.pallas import tpu as pltpu
from jax.experimental.pallas import tpu_sc as plsc

_BT = 4096


def _table_kernel(emb_ref, w_ref, bpad_ref, a_ref, carry_ref):
    i = pl.program_id(0)

    @pl.when(i == 0)
    def _():
        carry_ref[...] = jnp.zeros_like(carry_ref)

    h = emb_ref.shape[1]
    emb = emb_ref[...]
    nd = (((1,), (1,)), ((), ()))
    p1 = lax.dot_general(emb, w_ref[:, 0:h], nd,
                         preferred_element_type=jnp.float32)
    p2 = lax.dot_general(emb, w_ref[:, h : 2 * h], nd,
                         preferred_element_type=jnp.float32)
    p3 = lax.dot_general(emb, w_ref[:, 2 * h : 3 * h], nd,
                         preferred_element_type=jnp.float32)
    nl = p1.shape[1]
    bt = p1.shape[0]
    sb = 128
    r = lax.broadcasted_iota(jnp.int32, (sb, sb), 0)
    c = lax.broadcasted_iota(jnp.int32, (sb, sb), 1)
    tri = (r >= c).astype(jnp.float32)
    a_ref[:, 0:nl] = p1 + bpad_ref[0:1, 0:nl]
    a_ref[:, 64 : 64 + nl] = p2
    off = carry_ref[0:1, 0:nl]
    for q in range(bt // sb):
        p3q = p3[q * sb : (q + 1) * sb, :]
        cq = jnp.dot(tri, p3q, preferred_element_type=jnp.float32) + off
        a_ref[q * sb : (q + 1) * sb, 16 : 16 + nl] = p3q - cq
        a_ref[q * sb : (q + 1) * sb, 80 : 80 + nl] = cq
        off = cq[sb - 1 :, :]
    carry_ref[0:1, 0:nl] = off


def _build_table(token_emb, W, bpad):
    t, h = token_emb.shape
    nl = W.shape[0]
    grid = t // _BT
    return pl.pallas_call(
        _table_kernel,
        grid=(grid,),
        in_specs=[
            pl.BlockSpec((_BT, h), lambda i: (i, 0)),
            pl.BlockSpec((nl, 3 * h), lambda i: (0, 0)),
            pl.BlockSpec((8, 128), lambda i: (0, 0)),
        ],
        out_specs=pl.BlockSpec((_BT, 128), lambda i: (i, 0)),
        out_shape=jax.ShapeDtypeStruct((t, 128), jnp.float32),
        scratch_shapes=[pltpu.VMEM((8, 128), jnp.float32)],
        compiler_params=pltpu.CompilerParams(
            dimension_semantics=("arbitrary",)
        ),
    )(token_emb, W, bpad)


_SB = 128


def _make_sc_combine(n, num_logits):
    info = plsc.get_sparse_core_info()
    nc, ns = info.num_cores, info.num_subcores
    nw = nc * ns
    per_w = n // nw
    k_steps = per_w // _SB
    mesh = plsc.VectorSubcoreMesh(core_axis_name="c", subcore_axis_name="s")

    @functools.partial(
        pl.kernel,
        mesh=mesh,
        out_type=jax.ShapeDtypeStruct((n, 128), jnp.float32),
        scratch_types=[
            pltpu.VMEM((per_w,), jnp.int32),
            pltpu.VMEM((per_w,), jnp.int32),
            pltpu.VMEM((per_w,), jnp.float32),
            [pltpu.VMEM((_SB, 128), jnp.float32)] * 2,
            [pltpu.VMEM((_SB, 128), jnp.float32)] * 2,
            [pltpu.VMEM((_SB, 128), jnp.float32)] * 2,
            [pltpu.SemaphoreType.DMA] * 2,
            [pltpu.SemaphoreType.DMA] * 2,
        ],
        compiler_params=pltpu.CompilerParams(use_tc_tiling_on_sc=False),
    )
    def sc_combine(ta, spans_t, out_hbm, sv, em1, invr, ar, br,
                   outv, sem_a, sem_b):
        wid = lax.axis_index("s") * nc + lax.axis_index("c")

        pltpu.sync_copy(spans_t.at[0, pl.ds(wid * per_w, per_w)], sv)
        pltpu.sync_copy(spans_t.at[1, pl.ds(wid * per_w, per_w)], em1)
        for g in range(per_w // 16):
            svv = sv[pl.ds(g * 16, 16)]
            evv = em1[pl.ds(g * 16, 16)]
            em1[pl.ds(g * 16, 16)] = evv - 1
            invr[pl.ds(g * 16, 16)] = 1.0 / (evv - svv).astype(jnp.float32)

        def stage(k, p):
            pltpu.async_copy(
                ta.at[sv.at[pl.ds(k * _SB, _SB)]], ar[p], sem_a[p]
            )
            pltpu.async_copy(
                ta.at[em1.at[pl.ds(k * _SB, _SB)]], br[p], sem_b[p]
            )

        def finish(k, p):
            base = wid * per_w + k * _SB
            pltpu.make_async_copy(
                ta.at[sv.at[pl.ds(k * _SB, _SB)]], ar[p], sem_a[p]
            ).wait()
            pltpu.make_async_copy(
                ta.at[em1.at[pl.ds(k * _SB, _SB)]], br[p], sem_b[p]
            ).wait()
            for g in range(_SB // 16):
                invv = invr[pl.ds(k * _SB + g * 16, 16)]
                for u in range(16):
                    i = g * 16 + u
                    alo = ar[p][i, pl.ds(0, 16)]
                    ahi = ar[p][i, pl.ds(16, 16)]
                    blo = br[p][i, pl.ds(64, 16)]
                    bhi = br[p][i, pl.ds(80, 16)]
                    outv[p][i, pl.ds(0, 16)] = (
                        alo + blo + (bhi + ahi) * invv[u]
                    )
            pltpu.sync_copy(outv[p], out_hbm.at[pl.ds(base, _SB)])

        stage(0, 0)

        def body(g, carry):
            k0 = 2 * g
            stage(k0 + 1, 1)
            finish(k0, 0)

            @pl.when(g < k_steps // 2 - 1)
            def _():
                stage(k0 + 2, 0)

            finish(k0 + 1, 1)
            return carry

        lax.fori_loop(0, k_steps // 2, body, 0)

    return sc_combine


def kernel(token_emb, spans, W, b):
    t, h = token_emb.shape
    n = spans.shape[0]
    num_logits = W.shape[0]

    bpad = jnp.zeros((8, 128), jnp.float32).at[0, 0:num_logits].set(b)

    tab = _build_table(token_emb, W.astype(jnp.float32), bpad)

    spans_t = spans.astype(jnp.int32).T

    out = _make_sc_combine(n, num_logits)(tab, spans_t)
    return out[:, 0:num_logits]


# speedup vs baseline: 1.1519x; 1.0060x over previous
import functools

import jax
import jax.numpy as jnp
from jax import lax
from jax.experimental import pallas as pl
from jax.experimental.pallas import tpu as pltpu
from jax.experimental.pallas import tpu_sc as plsc

_BT = 4096


def _table_kernel(emb_ref, w_ref, bpad_ref, a_ref, carry_ref):
    i = pl.program_id(0)

    @pl.when(i == 0)
    def _():
        carry_ref[...] = jnp.zeros_like(carry_ref)

    h = emb_ref.shape[1]
    emb = emb_ref[...]
    nd = (((1,), (1,)), ((), ()))
    p1 = lax.dot_general(emb, w_ref[:, 0:h], nd,
                         preferred_element_type=jnp.float32)
    p2 = lax.dot_general(emb, w_ref[:, h : 2 * h], nd,
                         preferred_element_type=jnp.float32)
    p3 = lax.dot_general(emb, w_ref[:, 2 * h : 3 * h], nd,
                         preferred_element_type=jnp.float32)
    nl = p1.shape[1]
    bt = p1.shape[0]
    sb = 128
    r = lax.broadcasted_iota(jnp.int32, (sb, sb), 0)
    c = lax.broadcasted_iota(jnp.int32, (sb, sb), 1)
    tri = (r >= c).astype(jnp.float32)
    a_ref[:, 0:nl] = p1 + bpad_ref[0:1, 0:nl]
    a_ref[:, 64 : 64 + nl] = p2
    off = carry_ref[0:1, 0:nl]
    for q in range(bt // sb):
        p3q = p3[q * sb : (q + 1) * sb, :]
        cq = jnp.dot(tri, p3q, preferred_element_type=jnp.float32) + off
        a_ref[q * sb : (q + 1) * sb, 16 : 16 + nl] = p3q - cq
        a_ref[q * sb : (q + 1) * sb, 80 : 80 + nl] = cq
        off = cq[sb - 1 :, :]
    carry_ref[0:1, 0:nl] = off


def _build_table(token_emb, W, bpad):
    t, h = token_emb.shape
    nl = W.shape[0]
    grid = t // _BT
    return pl.pallas_call(
        _table_kernel,
        grid=(grid,),
        in_specs=[
            pl.BlockSpec((_BT, h), lambda i: (i, 0)),
            pl.BlockSpec((nl, 3 * h), lambda i: (0, 0)),
            pl.BlockSpec((8, 128), lambda i: (0, 0)),
        ],
        out_specs=pl.BlockSpec((_BT, 128), lambda i: (i, 0)),
        out_shape=jax.ShapeDtypeStruct((t, 128), jnp.float32),
        scratch_shapes=[pltpu.VMEM((8, 128), jnp.float32)],
        compiler_params=pltpu.CompilerParams(
            dimension_semantics=("arbitrary",)
        ),
    )(token_emb, W, bpad)


_SB = 128


def _make_sc_combine(n, num_logits):
    info = plsc.get_sparse_core_info()
    nc, ns = info.num_cores, info.num_subcores
    nw = nc * ns
    per_w = n // nw
    k_steps = per_w // _SB
    mesh = plsc.VectorSubcoreMesh(core_axis_name="c", subcore_axis_name="s")

    @functools.partial(
        pl.kernel,
        mesh=mesh,
        out_type=jax.ShapeDtypeStruct((n, 128), jnp.float32),
        scratch_types=[
            pltpu.VMEM((per_w,), jnp.int32),
            pltpu.VMEM((per_w,), jnp.int32),
            pltpu.VMEM((per_w,), jnp.float32),
            [pltpu.VMEM((_SB, 128), jnp.float32)] * 2,
            [pltpu.VMEM((_SB, 128), jnp.float32)] * 2,
            [pltpu.VMEM((_SB, 128), jnp.float32)] * 2,
            [pltpu.SemaphoreType.DMA] * 2,
            [pltpu.SemaphoreType.DMA] * 2,
            [pltpu.SemaphoreType.DMA] * 2,
        ],
        compiler_params=pltpu.CompilerParams(use_tc_tiling_on_sc=False),
    )
    def sc_combine(ta, spans_t, out_hbm, sv, em1, invr, ar, br,
                   outv, sem_a, sem_b, sem_o):
        wid = lax.axis_index("s") * nc + lax.axis_index("c")

        pltpu.sync_copy(spans_t.at[0, pl.ds(wid * per_w, per_w)], sv)
        pltpu.sync_copy(spans_t.at[1, pl.ds(wid * per_w, per_w)], em1)
        for g in range(per_w // 16):
            svv = sv[pl.ds(g * 16, 16)]
            evv = em1[pl.ds(g * 16, 16)]
            em1[pl.ds(g * 16, 16)] = evv - 1
            invr[pl.ds(g * 16, 16)] = 1.0 / (evv - svv).astype(jnp.float32)

        def stage(k, p):
            pltpu.async_copy(
                ta.at[sv.at[pl.ds(k * _SB, _SB)]], ar[p], sem_a[p]
            )
            pltpu.async_copy(
                ta.at[em1.at[pl.ds(k * _SB, _SB)]], br[p], sem_b[p]
            )

        def finish(k, p):
            base = wid * per_w + k * _SB
            pltpu.make_async_copy(
                ta.at[sv.at[pl.ds(k * _SB, _SB)]], ar[p], sem_a[p]
            ).wait()
            pltpu.make_async_copy(
                ta.at[em1.at[pl.ds(k * _SB, _SB)]], br[p], sem_b[p]
            ).wait()

            @pl.when(k >= 2)
            def _():
                pltpu.make_async_copy(
                    outv[p],
                    out_hbm.at[pl.ds(base - 2 * _SB, _SB)],
                    sem_o[p],
                ).wait()

            for g in range(_SB // 16):
                invv = invr[pl.ds(k * _SB + g * 16, 16)]
                for u in range(16):
                    i = g * 16 + u
                    alo = ar[p][i, pl.ds(0, 16)]
                    ahi = ar[p][i, pl.ds(16, 16)]
                    blo = br[p][i, pl.ds(64, 16)]
                    bhi = br[p][i, pl.ds(80, 16)]
                    outv[p][i, pl.ds(0, 16)] = (
                        alo + blo + (bhi + ahi) * invv[u]
                    )
            pltpu.async_copy(
                outv[p], out_hbm.at[pl.ds(base, _SB)], sem_o[p]
            )

        stage(0, 0)

        def body(g, carry):
            k0 = 2 * g
            stage(k0 + 1, 1)
            finish(k0, 0)

            @pl.when(g < k_steps // 2 - 1)
            def _():
                stage(k0 + 2, 0)

            finish(k0 + 1, 1)
            return carry

        lax.fori_loop(0, k_steps // 2, body, 0)

        for p, k in ((0, k_steps - 2), (1, k_steps - 1)):
            base = wid * per_w + k * _SB
            pltpu.make_async_copy(
                outv[p], out_hbm.at[pl.ds(base, _SB)], sem_o[p]
            ).wait()

    return sc_combine


def kernel(token_emb, spans, W, b):
    t, h = token_emb.shape
    n = spans.shape[0]
    num_logits = W.shape[0]

    bpad = jnp.zeros((8, 128), jnp.float32).at[0, 0:num_logits].set(b)

    tab = _build_table(token_emb, W.astype(jnp.float32), bpad)

    spans_t = spans.astype(jnp.int32).T

    out = _make_sc_combine(n, num_logits)(tab, spans_t)
    return out[:, 0:num_logits]
